# baseline (speedup 1.0000x reference)
import jax
import jax.numpy as jnp
from jax import lax
from jax.experimental import pallas as pl
from jax.experimental.pallas import tpu as pltpu
from jax.experimental.pallas import tpu_sc as plsc

_BV = 2048


def _sc_gather_body(table_hbm, idx_hbm, out_hbm, idx_v, rows_v, sem, *,
                    n_cores, b_per_w):
    wid = lax.axis_index("s") * n_cores + lax.axis_index("c")
    base = wid * b_per_w
    pltpu.sync_copy(idx_hbm.at[pl.ds(base, b_per_w)], idx_v)
    pltpu.async_copy(table_hbm.at[idx_v], rows_v, sem).wait()
    pltpu.sync_copy(rows_v, out_hbm.at[pl.ds(base, b_per_w)])


def _sc_gather(table, idx_flat):
    rows, d = idx_flat.shape[0], table.shape[1]
    info = plsc.get_sparse_core_info()
    n_workers = info.num_cores * info.num_subcores
    b_per_w = rows // n_workers
    import functools
    body = functools.partial(_sc_gather_body, n_cores=info.num_cores,
                             b_per_w=b_per_w)
    mesh = plsc.VectorSubcoreMesh(core_axis_name="c", subcore_axis_name="s")
    return pl.kernel(
        body,
        mesh=mesh,
        out_type=jax.ShapeDtypeStruct((rows, d), jnp.float32),
        scratch_types=[
            pltpu.VMEM((b_per_w,), jnp.int32),
            pltpu.VMEM((b_per_w, d), jnp.float32),
            pltpu.SemaphoreType.DMA,
        ],
        compiler_params=pltpu.CompilerParams(use_tc_tiling_on_sc=False),
    )(table, idx_flat)


def _gelu_exact(v):
    return 0.5 * v * (1.0 + lax.erf(v * jnp.float32(0.7071067811865476)))


def _mlp_body(h_ref, W1_ref, b1_ref, W2_ref, b2_ref, W3_ref, b3_ref,
              out_ref, h2_ref):
    @pl.when(pl.program_id(0) == 0)
    def _():
        a1 = jnp.dot(h_ref[...], W1_ref[...],
                     preferred_element_type=jnp.float32) + b1_ref[...]
        g1 = _gelu_exact(a1)
        a2 = jnp.dot(g1, W2_ref[...],
                     preferred_element_type=jnp.float32) + b2_ref[...]
        h2_ref[...] = _gelu_exact(a2)
    out_ref[...] = jnp.dot(h2_ref[...], W3_ref[...],
                           preferred_element_type=jnp.float32) + b3_ref[...]


def kernel(x, emb, W1, b1, W2, b2, W3, b3):
    batch, ctx = x.shape
    vocab, d = emb.shape
    hidden = W1.shape[1]
    rows = batch * ctx

    gathered = _sc_gather(emb, x.reshape(rows))
    h = gathered.reshape(batch, ctx * d)

    grid = pl.cdiv(vocab, _BV)
    out = pl.pallas_call(
        _mlp_body,
        grid=(grid,),
        in_specs=[
            pl.BlockSpec((batch, ctx * d), lambda i: (0, 0)),
            pl.BlockSpec((ctx * d, hidden), lambda i: (0, 0)),
            pl.BlockSpec((1, hidden), lambda i: (0, 0)),
            pl.BlockSpec((hidden, hidden), lambda i: (0, 0)),
            pl.BlockSpec((1, hidden), lambda i: (0, 0)),
            pl.BlockSpec((hidden, _BV), lambda i: (0, i)),
            pl.BlockSpec((1, _BV), lambda i: (0, i)),
        ],
        out_specs=pl.BlockSpec((batch, _BV), lambda i: (0, i)),
        out_shape=jax.ShapeDtypeStruct((batch, vocab), jnp.float32),
        scratch_shapes=[pltpu.VMEM((batch, hidden), jnp.float32)],
        compiler_params=pltpu.CompilerParams(
            dimension_semantics=("arbitrary",)),
    )(h, W1, b1.reshape(1, hidden), W2, b2.reshape(1, hidden), W3,
      b3.reshape(1, vocab))
    return out


# speedup vs baseline: 1.0046x; 1.0046x over previous
import jax
import jax.numpy as jnp
from jax import lax
from jax.experimental import pallas as pl
from jax.experimental.pallas import tpu as pltpu
from jax.experimental.pallas import tpu_sc as plsc

_BV = 4096


def _sc_gather_body(table_hbm, idx_hbm, out_hbm, idx_v, rows_v, sem, *,
                    n_cores, b_per_w):
    wid = lax.axis_index("s") * n_cores + lax.axis_index("c")
    base = wid * b_per_w
    pltpu.sync_copy(idx_hbm.at[pl.ds(base, b_per_w)], idx_v)
    pltpu.async_copy(table_hbm.at[idx_v], rows_v, sem).wait()
    pltpu.sync_copy(rows_v, out_hbm.at[pl.ds(base, b_per_w)])


def _sc_gather(table, idx_flat):
    rows, d = idx_flat.shape[0], table.shape[1]
    info = plsc.get_sparse_core_info()
    n_workers = info.num_cores * info.num_subcores
    b_per_w = rows // n_workers
    import functools
    body = functools.partial(_sc_gather_body, n_cores=info.num_cores,
                             b_per_w=b_per_w)
    mesh = plsc.VectorSubcoreMesh(core_axis_name="c", subcore_axis_name="s")
    return pl.kernel(
        body,
        mesh=mesh,
        out_type=jax.ShapeDtypeStruct((rows, d), jnp.float32),
        scratch_types=[
            pltpu.VMEM((b_per_w,), jnp.int32),
            pltpu.VMEM((b_per_w, d), jnp.float32),
            pltpu.SemaphoreType.DMA,
        ],
        compiler_params=pltpu.CompilerParams(use_tc_tiling_on_sc=False),
    )(table, idx_flat)


def _gelu_exact(v):
    return 0.5 * v * (1.0 + lax.erf(v * jnp.float32(0.7071067811865476)))


def _mlp_body(h_ref, W1_ref, b1_ref, W2_ref, b2_ref, W3_ref, b3_ref,
              out_ref, h2_ref):
    @pl.when(pl.program_id(0) == 0)
    def _():
        a1 = jnp.dot(h_ref[...], W1_ref[...],
                     preferred_element_type=jnp.float32) + b1_ref[...]
        g1 = _gelu_exact(a1)
        a2 = jnp.dot(g1, W2_ref[...],
                     preferred_element_type=jnp.float32) + b2_ref[...]
        h2_ref[...] = _gelu_exact(a2)
    out_ref[...] = jnp.dot(h2_ref[...], W3_ref[...],
                           preferred_element_type=jnp.float32) + b3_ref[...]


def kernel(x, emb, W1, b1, W2, b2, W3, b3):
    batch, ctx = x.shape
    vocab, d = emb.shape
    hidden = W1.shape[1]
    rows = batch * ctx

    gathered = _sc_gather(emb, x.reshape(rows))
    h = gathered.reshape(batch, ctx * d)

    grid = pl.cdiv(vocab, _BV)
    out = pl.pallas_call(
        _mlp_body,
        grid=(grid,),
        in_specs=[
            pl.BlockSpec((batch, ctx * d), lambda i: (0, 0)),
            pl.BlockSpec((ctx * d, hidden), lambda i: (0, 0)),
            pl.BlockSpec((1, hidden), lambda i: (0, 0)),
            pl.BlockSpec((hidden, hidden), lambda i: (0, 0)),
            pl.BlockSpec((1, hidden), lambda i: (0, 0)),
            pl.BlockSpec((hidden, _BV), lambda i: (0, i)),
            pl.BlockSpec((1, _BV), lambda i: (0, i)),
        ],
        out_specs=pl.BlockSpec((batch, _BV), lambda i: (0, i)),
        out_shape=jax.ShapeDtypeStruct((batch, vocab), jnp.float32),
        scratch_shapes=[pltpu.VMEM((batch, hidden), jnp.float32)],
        compiler_params=pltpu.CompilerParams(
            dimension_semantics=("arbitrary",),
            vmem_limit_bytes=100 * 1024 * 1024),
    )(h, W1, b1.reshape(1, hidden), W2, b2.reshape(1, hidden), W3,
      b3.reshape(1, vocab))
    return out


# speedup vs baseline: 2.1697x; 2.1596x over previous
import jax
import jax.numpy as jnp
from jax import lax
from jax.experimental import pallas as pl
from jax.experimental.pallas import tpu as pltpu
from jax.experimental.pallas import tpu_sc as plsc

_BV = 4096


def _sc_gather_body(table_hbm, idx_hbm, out_hbm, idx_v, rows_v, sem, *,
                    n_cores, b_per_w):
    wid = lax.axis_index("s") * n_cores + lax.axis_index("c")
    base = wid * b_per_w
    pltpu.sync_copy(idx_hbm.at[pl.ds(base, b_per_w)], idx_v)
    pltpu.async_copy(table_hbm.at[idx_v], rows_v, sem).wait()
    pltpu.sync_copy(rows_v, out_hbm.at[pl.ds(base, b_per_w)])


def _sc_gather(table, idx_flat):
    rows, d = idx_flat.shape[0], table.shape[1]
    info = plsc.get_sparse_core_info()
    n_workers = info.num_cores * info.num_subcores
    b_per_w = rows // n_workers
    import functools
    body = functools.partial(_sc_gather_body, n_cores=info.num_cores,
                             b_per_w=b_per_w)
    mesh = plsc.VectorSubcoreMesh(core_axis_name="c", subcore_axis_name="s")
    return pl.kernel(
        body,
        mesh=mesh,
        out_type=jax.ShapeDtypeStruct((rows, d), jnp.float32),
        scratch_types=[
            pltpu.VMEM((b_per_w,), jnp.int32),
            pltpu.VMEM((b_per_w, d), jnp.float32),
            pltpu.SemaphoreType.DMA,
        ],
        compiler_params=pltpu.CompilerParams(use_tc_tiling_on_sc=False),
    )(table, idx_flat)


def _gelu_exact(v):
    return 0.5 * v * (1.0 + lax.erf(v * jnp.float32(0.7071067811865476)))


def _mlp_body(h_ref, W1_ref, b1_ref, W2_ref, b2_ref, W3_ref, b3_ref,
              out_ref, h2t_ref):
    @pl.when(pl.program_id(0) == 0)
    def _():
        a1 = jnp.dot(h_ref[...], W1_ref[...],
                     preferred_element_type=jnp.float32) + b1_ref[...]
        g1 = _gelu_exact(a1)
        a2 = jnp.dot(g1, W2_ref[...],
                     preferred_element_type=jnp.float32) + b2_ref[...]
        h2t_ref[...] = _gelu_exact(a2).T
    out_ref[...] = lax.dot_general(
        W3_ref[...], h2t_ref[...],
        dimension_numbers=(((0,), (0,)), ((), ())),
        preferred_element_type=jnp.float32) + b3_ref[...]


def kernel(x, emb, W1, b1, W2, b2, W3, b3):
    batch, ctx = x.shape
    vocab, d = emb.shape
    hidden = W1.shape[1]
    rows = batch * ctx

    gathered = _sc_gather(emb, x.reshape(rows))
    h = gathered.reshape(batch, ctx * d)

    grid = pl.cdiv(vocab, _BV)
    out_t = pl.pallas_call(
        _mlp_body,
        grid=(grid,),
        in_specs=[
            pl.BlockSpec((batch, ctx * d), lambda i: (0, 0)),
            pl.BlockSpec((ctx * d, hidden), lambda i: (0, 0)),
            pl.BlockSpec((1, hidden), lambda i: (0, 0)),
            pl.BlockSpec((hidden, hidden), lambda i: (0, 0)),
            pl.BlockSpec((1, hidden), lambda i: (0, 0)),
            pl.BlockSpec((hidden, _BV), lambda i: (0, i)),
            pl.BlockSpec((_BV, 1), lambda i: (i, 0)),
        ],
        out_specs=pl.BlockSpec((_BV, batch), lambda i: (i, 0)),
        out_shape=jax.ShapeDtypeStruct((vocab, batch), jnp.float32),
        scratch_shapes=[pltpu.VMEM((hidden, batch), jnp.float32)],
        compiler_params=pltpu.CompilerParams(
            dimension_semantics=("arbitrary",),
            vmem_limit_bytes=100 * 1024 * 1024),
    )(h, W1, b1.reshape(1, hidden), W2, b2.reshape(1, hidden), W3,
      b3.reshape(vocab, 1))
    return out_t.T


# speedup vs baseline: 2.5229x; 1.1628x over previous
import functools

import jax
import jax.numpy as jnp
from jax import lax
from jax.experimental import pallas as pl
from jax.experimental.pallas import tpu as pltpu
from jax.experimental.pallas import tpu_sc as plsc

_BV = 4096
_CHUNK = 128


def _sc_gather_body(flat_hbm, idx_hbm, out_hbm, idx_v, offs_v, rows_v, sem,
                    *, n_cores, b_per_w, vocab, d):
    wid = lax.axis_index("s") * n_cores + lax.axis_index("c")
    base = wid * b_per_w
    pltpu.sync_copy(idx_hbm.at[pl.ds(base, b_per_w)], idx_v)
    lane = lax.iota(jnp.int32, 16)
    n_groups = b_per_w // 16
    words = b_per_w * d
    n_chunks = words // _CHUNK
    for g in range(n_groups):
        v = idx_v[pl.ds(g * 16, 16)]
        for k in range(d):
            plsc.store_scatter(
                offs_v, [g * 16 * d + lane * d + k], v + k * vocab)
    copies = []
    for c in range(n_chunks):
        copies.append(pltpu.async_copy(
            flat_hbm.at[offs_v.at[pl.ds(c * _CHUNK, _CHUNK)]],
            rows_v.at[c], sem))
    for c in copies:
        c.wait()
    pltpu.sync_copy(rows_v, out_hbm.at[wid])


def _sc_gather(flat_table, idx_flat, vocab, d):
    rows = idx_flat.shape[0]
    info = plsc.get_sparse_core_info()
    n_workers = info.num_cores * info.num_subcores
    b_per_w = rows // n_workers
    words = b_per_w * d
    body = functools.partial(_sc_gather_body, n_cores=info.num_cores,
                             b_per_w=b_per_w, vocab=vocab, d=d)
    mesh = plsc.VectorSubcoreMesh(core_axis_name="c", subcore_axis_name="s")
    out = pl.kernel(
        body,
        mesh=mesh,
        out_type=jax.ShapeDtypeStruct((n_workers, words // _CHUNK, _CHUNK),
                                      jnp.float32),
        scratch_types=[
            pltpu.VMEM((b_per_w,), jnp.int32),
            pltpu.VMEM((words,), jnp.int32),
            pltpu.VMEM((words // _CHUNK, _CHUNK), jnp.float32),
            pltpu.SemaphoreType.DMA,
        ],
        compiler_params=pltpu.CompilerParams(use_tc_tiling_on_sc=False,
                                             needs_layout_passes=False),
    )(flat_table, idx_flat)
    return out.reshape(rows, d)


def _gelu_exact(v):
    return 0.5 * v * (1.0 + lax.erf(v * jnp.float32(0.7071067811865476)))


def _mlp_body(h_ref, W1_ref, b1_ref, W2_ref, b2_ref, W3_ref, b3_ref,
              out_ref, h2t_ref):
    @pl.when(pl.program_id(0) == 0)
    def _():
        a1 = jnp.dot(h_ref[...], W1_ref[...],
                     preferred_element_type=jnp.float32) + b1_ref[...]
        g1 = _gelu_exact(a1)
        a2 = jnp.dot(g1, W2_ref[...],
                     preferred_element_type=jnp.float32) + b2_ref[...]
        h2t_ref[...] = _gelu_exact(a2).T.astype(jnp.bfloat16)
    out_ref[...] = lax.dot_general(
        W3_ref[...].astype(jnp.bfloat16), h2t_ref[...],
        dimension_numbers=(((0,), (0,)), ((), ())),
        preferred_element_type=jnp.float32) + b3_ref[...]


def kernel(x, emb, W1, b1, W2, b2, W3, b3):
    batch, ctx = x.shape
    vocab, d = emb.shape
    hidden = W1.shape[1]
    rows = batch * ctx

    flat_table = emb.T.reshape(vocab * d)
    gathered = _sc_gather(flat_table, x.reshape(rows), vocab, d)
    h = gathered.reshape(batch, ctx * d)

    grid = pl.cdiv(vocab, _BV)
    out_t = pl.pallas_call(
        _mlp_body,
        grid=(grid,),
        in_specs=[
            pl.BlockSpec((batch, ctx * d), lambda i: (0, 0)),
            pl.BlockSpec((ctx * d, hidden), lambda i: (0, 0)),
            pl.BlockSpec((1, hidden), lambda i: (0, 0)),
            pl.BlockSpec((hidden, hidden), lambda i: (0, 0)),
            pl.BlockSpec((1, hidden), lambda i: (0, 0)),
            pl.BlockSpec((hidden, _BV), lambda i: (0, i)),
            pl.BlockSpec((_BV, 1), lambda i: (i, 0)),
        ],
        out_specs=pl.BlockSpec((_BV, batch), lambda i: (i, 0)),
        out_shape=jax.ShapeDtypeStruct((vocab, batch), jnp.float32),
        scratch_shapes=[pltpu.VMEM((hidden, batch), jnp.bfloat16)],
        compiler_params=pltpu.CompilerParams(
            dimension_semantics=("arbitrary",),
            vmem_limit_bytes=100 * 1024 * 1024),
    )(h, W1, b1.reshape(1, hidden), W2, b2.reshape(1, hidden), W3,
      b3.reshape(vocab, 1))
    return out_t.T


# speedup vs baseline: 3.3462x; 1.3263x over previous
import functools

import jax
import jax.numpy as jnp
from jax import lax
from jax.experimental import pallas as pl
from jax.experimental.pallas import tpu as pltpu
from jax.experimental.pallas import tpu_sc as plsc

_BV = 4096
_CHUNK = 128


def _sc_gather_body(flat_hbm, idx_hbm, out_hbm, idx_v, offs_v, rows_v, sem,
                    *, n_cores, b_per_w, vocab, d):
    wid = lax.axis_index("s") * n_cores + lax.axis_index("c")
    base = wid * b_per_w
    pltpu.sync_copy(idx_hbm.at[pl.ds(base, b_per_w)], idx_v)
    lane = lax.iota(jnp.int32, 16)
    n_groups = b_per_w // 16
    words = b_per_w * d
    n_chunks = words // _CHUNK
    for g in range(n_groups):
        v = idx_v[pl.ds(g * 16, 16)]
        for k in range(d):
            plsc.store_scatter(
                offs_v, [g * 16 * d + lane * d + k], v + k * vocab)
    copies = []
    for c in range(n_chunks):
        copies.append(pltpu.async_copy(
            flat_hbm.at[offs_v.at[pl.ds(c * _CHUNK, _CHUNK)]],
            rows_v.at[c], sem))
    for c in copies:
        c.wait()
    pltpu.sync_copy(rows_v, out_hbm.at[wid])


def _sc_gather(flat_table, idx_flat, vocab, d):
    rows = idx_flat.shape[0]
    info = plsc.get_sparse_core_info()
    n_workers = info.num_cores * info.num_subcores
    b_per_w = rows // n_workers
    words = b_per_w * d
    body = functools.partial(_sc_gather_body, n_cores=info.num_cores,
                             b_per_w=b_per_w, vocab=vocab, d=d)
    mesh = plsc.VectorSubcoreMesh(core_axis_name="c", subcore_axis_name="s")
    out = pl.kernel(
        body,
        mesh=mesh,
        out_type=jax.ShapeDtypeStruct((n_workers, words // _CHUNK, _CHUNK),
                                      jnp.float32),
        scratch_types=[
            pltpu.VMEM((b_per_w,), jnp.int32),
            pltpu.VMEM((words,), jnp.int32),
            pltpu.VMEM((words // _CHUNK, _CHUNK), jnp.float32),
            pltpu.SemaphoreType.DMA,
        ],
        compiler_params=pltpu.CompilerParams(use_tc_tiling_on_sc=False,
                                             needs_layout_passes=False),
    )(flat_table, idx_flat)
    return out.reshape(rows, d)


def _gelu_exact(v):
    return 0.5 * v * (1.0 + lax.erf(v * jnp.float32(0.7071067811865476)))


def _mlp_body(h_ref, W1_ref, b1_ref, W2_ref, b2_ref, W3_ref, b3_ref,
              out_ref, h2t_ref):
    @pl.when(pl.program_id(0) == 0)
    def _():
        a1 = jnp.dot(h_ref[...], W1_ref[...],
                     preferred_element_type=jnp.float32) + b1_ref[...]
        g1 = _gelu_exact(a1)
        a2 = jnp.dot(g1, W2_ref[...],
                     preferred_element_type=jnp.float32) + b2_ref[...]
        h2t_ref[...] = _gelu_exact(a2).T.astype(jnp.bfloat16)
    batch = out_ref.shape[1]
    bias = lax.dot_general(
        b3_ref[...], jnp.ones((1, batch), jnp.float32),
        dimension_numbers=(((0,), (0,)), ((), ())),
        preferred_element_type=jnp.float32)
    out_ref[...] = lax.dot_general(
        W3_ref[...].astype(jnp.bfloat16), h2t_ref[...],
        dimension_numbers=(((0,), (0,)), ((), ())),
        preferred_element_type=jnp.float32) + bias


def kernel(x, emb, W1, b1, W2, b2, W3, b3):
    batch, ctx = x.shape
    vocab, d = emb.shape
    hidden = W1.shape[1]
    rows = batch * ctx

    flat_table = emb.T.reshape(vocab * d)
    gathered = _sc_gather(flat_table, x.reshape(rows), vocab, d)
    h = gathered.reshape(batch, ctx * d)

    grid = pl.cdiv(vocab, _BV)
    out_t = pl.pallas_call(
        _mlp_body,
        grid=(grid,),
        in_specs=[
            pl.BlockSpec((batch, ctx * d), lambda i: (0, 0)),
            pl.BlockSpec((ctx * d, hidden), lambda i: (0, 0)),
            pl.BlockSpec((1, hidden), lambda i: (0, 0)),
            pl.BlockSpec((hidden, hidden), lambda i: (0, 0)),
            pl.BlockSpec((1, hidden), lambda i: (0, 0)),
            pl.BlockSpec((hidden, _BV), lambda i: (0, i)),
            pl.BlockSpec((1, _BV), lambda i: (0, i)),
        ],
        out_specs=pl.BlockSpec((_BV, batch), lambda i: (i, 0)),
        out_shape=jax.ShapeDtypeStruct((vocab, batch), jnp.float32),
        scratch_shapes=[pltpu.VMEM((hidden, batch), jnp.bfloat16)],
        compiler_params=pltpu.CompilerParams(
            dimension_semantics=("arbitrary",),
            vmem_limit_bytes=60 * 1024 * 1024),
    )(h, W1, b1.reshape(1, hidden), W2, b2.reshape(1, hidden), W3,
      b3.reshape(1, vocab))
    return out_t.T
